# baseline (speedup 1.0000x reference)
import functools

import jax
import jax.numpy as jnp
from jax import lax
from jax.experimental import pallas as pl
from jax.experimental.pallas import tpu as pltpu
from jax.experimental.pallas import tpu_sc as plsc

def _vreg_gather(vals, idx):
    return lax.gather(
        vals,
        idx[:, None],
        lax.GatherDimensionNumbers(
            offset_dims=(), collapsed_slice_dims=(0,), start_index_map=(0,)
        ),
        slice_sizes=(1,),
        mode=lax.GatherScatterMode.PROMISE_IN_BOUNDS,
    )


def _body(values_hbm, out_hbm, vals_v, out_v):
    wid = lax.axis_index("s") * 2 + lax.axis_index("c")

    @pl.when(wid == 0)
    def _():
        pltpu.sync_copy(values_hbm, vals_v.at[pl.ds(0, 4)])
        vals = vals_v[...]
        lane = lax.iota(jnp.int32, 16)
        src0 = jnp.where(
            lane == 2, 0, jnp.where(lane == 7, 1, jnp.where(lane == 13, 2, 0))
        )
        g0 = _vreg_gather(vals, src0)
        mask0 = (lane == 2) | (lane == 7) | (lane == 13)
        out_v[pl.ds(0, 16)] = jnp.where(mask0, g0, 0.0)
        g1 = _vreg_gather(vals, jnp.full((16,), 3, jnp.int32))
        out_v[pl.ds(16, 16)] = jnp.where(lane == 23 - 16, g1, 0.0)
        pltpu.sync_copy(out_v.at[pl.ds(0, 24)], out_hbm)


_scatter = pl.kernel(
    _body,
    out_type=jax.ShapeDtypeStruct((24,), jnp.float32),
    mesh=plsc.VectorSubcoreMesh(core_axis_name="c", subcore_axis_name="s"),
    scratch_types=[
        pltpu.VMEM((16,), jnp.float32),
        pltpu.VMEM((32,), jnp.float32),
    ],
)


@jax.jit
def kernel(values):
    return _scatter(values).reshape(4, 6)


# speedup vs baseline: 1.0685x; 1.0685x over previous
import functools

import jax
import jax.numpy as jnp
from jax import lax
from jax.experimental import pallas as pl
from jax.experimental.pallas import tpu as pltpu
from jax.experimental.pallas import tpu_sc as plsc

def _vreg_gather(vals, idx):
    return lax.gather(
        vals,
        idx[:, None],
        lax.GatherDimensionNumbers(
            offset_dims=(), collapsed_slice_dims=(0,), start_index_map=(0,)
        ),
        slice_sizes=(1,),
        mode=lax.GatherScatterMode.PROMISE_IN_BOUNDS,
    )


def _body(values_hbm, out_hbm, vals_v, out_v):
    wid = lax.axis_index("s") * 2 + lax.axis_index("c")

    @pl.when(wid == 0)
    def _():
        pltpu.sync_copy(values_hbm, vals_v.at[pl.ds(0, 4)])
        vals = vals_v[...]
        lane = lax.iota(jnp.int32, 16)
        src0 = jnp.where(
            lane == 2, 0, jnp.where(lane == 7, 1, jnp.where(lane == 13, 2, 0))
        )
        g0 = _vreg_gather(vals, src0)
        mask0 = (lane == 2) | (lane == 7) | (lane == 13)
        out_v[pl.ds(0, 16)] = jnp.where(mask0, g0, 0.0)
        g1 = _vreg_gather(vals, jnp.full((16,), 3, jnp.int32))
        out_v[pl.ds(16, 16)] = jnp.where(lane == 23 - 16, g1, 0.0)
        pltpu.sync_copy(out_v.at[pl.ds(0, 24)], out_hbm)


_scatter = pl.kernel(
    _body,
    out_type=jax.ShapeDtypeStruct((24,), jnp.float32),
    mesh=plsc.VectorSubcoreMesh(
        core_axis_name="c", subcore_axis_name="s", num_cores=1
    ),
    scratch_types=[
        pltpu.VMEM((16,), jnp.float32),
        pltpu.VMEM((32,), jnp.float32),
    ],
)


@jax.jit
def kernel(values):
    return _scatter(values).reshape(4, 6)
